# Initial kernel scaffold; baseline (speedup 1.0000x reference)
import jax
import jax.numpy as jnp
from jax.experimental import pallas as pl


def kernel(species, coordinates, conv_tensor):
    raise NotImplementedError("write your pallas kernel here")


# speedup vs baseline: 87.7308x; 87.7308x over previous
import functools

import jax
import jax.numpy as jnp
from jax import lax
from jax.experimental import pallas as pl
from jax.experimental.pallas import tpu as pltpu
from jax.experimental.pallas import tpu_sc as plsc

_NC, _NS, _L = 2, 16, 16
_NW = _NC * _NS
_LUT_PAD = 128


def _lut_kernel(total, species_hbm, conv_hbm, out_hbm, spec_v, conv_v, out_v):
    chunk = total // _NW
    wid = lax.axis_index("s") * _NC + lax.axis_index("c")
    base = wid * chunk
    pltpu.sync_copy(conv_hbm, conv_v)
    pltpu.sync_copy(species_hbm.at[pl.ds(base, chunk)], spec_v)

    def body(i, carry):
        off = i * _L
        idx = spec_v[pl.ds(off, _L)]
        out_v[pl.ds(off, _L)] = plsc.load_gather(conv_v, [idx])
        return carry

    lax.fori_loop(0, chunk // _L, body, 0)
    pltpu.sync_copy(out_v, out_hbm.at[pl.ds(base, chunk)])


def kernel(species, coordinates, conv_tensor):
    shape = species.shape
    flat = species.reshape(-1)
    total = flat.shape[0]
    chunk = total // _NW
    conv_pad = jnp.pad(conv_tensor, (0, _LUT_PAD - conv_tensor.shape[0]))
    mesh = plsc.VectorSubcoreMesh(
        core_axis_name="c", subcore_axis_name="s", num_cores=_NC,
        num_subcores=_NS)
    out = pl.kernel(
        functools.partial(_lut_kernel, total),
        out_type=jax.ShapeDtypeStruct((total,), jnp.int32),
        mesh=mesh,
        scratch_types=[
            pltpu.VMEM((chunk,), jnp.int32),
            pltpu.VMEM((_LUT_PAD,), jnp.int32),
            pltpu.VMEM((chunk,), jnp.int32),
        ],
        compiler_params=pltpu.CompilerParams(needs_layout_passes=False),
    )(flat, conv_pad)
    return (out.reshape(shape), coordinates)


# speedup vs baseline: 101.1579x; 1.1530x over previous
import functools

import jax
import jax.numpy as jnp
from jax import lax
from jax.experimental import pallas as pl
from jax.experimental.pallas import tpu as pltpu
from jax.experimental.pallas import tpu_sc as plsc

_NC, _NS, _L = 2, 16, 16
_NW = _NC * _NS
_LUT_PAD = 128


def _lut_kernel(total, species_hbm, conv_hbm, out_hbm, spec_v, conv_v, out_v):
    chunk = total // _NW
    wid = lax.axis_index("s") * _NC + lax.axis_index("c")
    base = wid * chunk
    pltpu.sync_copy(conv_hbm, conv_v)
    pltpu.sync_copy(species_hbm.at[pl.ds(base, chunk)], spec_v)

    @plsc.parallel_loop(0, chunk, step=_L, unroll=8)
    def _gather(off):
        idx = spec_v[pl.ds(off, _L)]
        out_v[pl.ds(off, _L)] = plsc.load_gather(conv_v, [idx])
    pltpu.sync_copy(out_v, out_hbm.at[pl.ds(base, chunk)])


def kernel(species, coordinates, conv_tensor):
    shape = species.shape
    flat = species.reshape(-1)
    total = flat.shape[0]
    chunk = total // _NW
    conv_pad = jnp.pad(conv_tensor, (0, _LUT_PAD - conv_tensor.shape[0]))
    mesh = plsc.VectorSubcoreMesh(
        core_axis_name="c", subcore_axis_name="s", num_cores=_NC,
        num_subcores=_NS)
    out = pl.kernel(
        functools.partial(_lut_kernel, total),
        out_type=jax.ShapeDtypeStruct((total,), jnp.int32),
        mesh=mesh,
        scratch_types=[
            pltpu.VMEM((chunk,), jnp.int32),
            pltpu.VMEM((_LUT_PAD,), jnp.int32),
            pltpu.VMEM((chunk,), jnp.int32),
        ],
        compiler_params=pltpu.CompilerParams(needs_layout_passes=False),
    )(flat, conv_pad)
    return (out.reshape(shape), coordinates)


# speedup vs baseline: 104.1160x; 1.0292x over previous
import functools

import jax
import jax.numpy as jnp
from jax import lax
from jax.experimental import pallas as pl
from jax.experimental.pallas import tpu as pltpu
from jax.experimental.pallas import tpu_sc as plsc

_NC, _NS, _L = 2, 16, 16
_NW = _NC * _NS
_LUT_PAD = 128


def _lut_kernel(total, species_hbm, conv_hbm, out_hbm, spec_v, conv_v, out_v,
                sem_c, sem_s):
    chunk = total // _NW
    wid = lax.axis_index("s") * _NC + lax.axis_index("c")
    base = wid * chunk
    cp_c = pltpu.async_copy(conv_hbm, conv_v, sem_c)
    cp_s = pltpu.async_copy(species_hbm.at[pl.ds(base, chunk)], spec_v, sem_s)
    cp_c.wait()
    cp_s.wait()

    @plsc.parallel_loop(0, chunk, step=_L, unroll=16)
    def _gather(off):
        idx = spec_v[pl.ds(off, _L)]
        out_v[pl.ds(off, _L)] = plsc.load_gather(conv_v, [idx])
    pltpu.sync_copy(out_v, out_hbm.at[pl.ds(base, chunk)])


def kernel(species, coordinates, conv_tensor):
    shape = species.shape
    flat = species.reshape(-1)
    total = flat.shape[0]
    chunk = total // _NW
    conv_pad = jnp.pad(conv_tensor, (0, _LUT_PAD - conv_tensor.shape[0]))
    mesh = plsc.VectorSubcoreMesh(
        core_axis_name="c", subcore_axis_name="s", num_cores=_NC,
        num_subcores=_NS)
    out = pl.kernel(
        functools.partial(_lut_kernel, total),
        out_type=jax.ShapeDtypeStruct((total,), jnp.int32),
        mesh=mesh,
        scratch_types=[
            pltpu.VMEM((chunk,), jnp.int32),
            pltpu.VMEM((_LUT_PAD,), jnp.int32),
            pltpu.VMEM((chunk,), jnp.int32),
            pltpu.SemaphoreType.DMA,
            pltpu.SemaphoreType.DMA,
        ],
        compiler_params=pltpu.CompilerParams(needs_layout_passes=False),
    )(flat, conv_pad)
    return (out.reshape(shape), coordinates)


# speedup vs baseline: 104.6144x; 1.0048x over previous
import functools

import jax
import jax.numpy as jnp
from jax import lax
from jax.experimental import pallas as pl
from jax.experimental.pallas import tpu as pltpu
from jax.experimental.pallas import tpu_sc as plsc

_NC, _NS, _L = 2, 16, 16
_NW = _NC * _NS
_LUT_PAD = 128


def _lut_kernel(total, species_hbm, conv_hbm, out_hbm, spec_v, conv_v, out_v,
                sem_c, sem_s):
    chunk = total // _NW
    wid = lax.axis_index("s") * _NC + lax.axis_index("c")
    base = wid * chunk
    half = chunk // 2
    cp_c = pltpu.async_copy(conv_hbm, conv_v, sem_c)
    cp_s0 = pltpu.async_copy(
        species_hbm.at[pl.ds(base, half)], spec_v.at[pl.ds(0, half)], sem_s)
    cp_s1 = pltpu.async_copy(
        species_hbm.at[pl.ds(base + half, half)],
        spec_v.at[pl.ds(half, half)], sem_s)
    cp_c.wait()
    cp_s0.wait()

    @plsc.parallel_loop(0, half, step=_L, unroll=16)
    def _gather0(off):
        idx = spec_v[pl.ds(off, _L)]
        out_v[pl.ds(off, _L)] = plsc.load_gather(conv_v, [idx])

    cp_o0 = pltpu.async_copy(
        out_v.at[pl.ds(0, half)], out_hbm.at[pl.ds(base, half)], sem_c)
    cp_s1.wait()

    @plsc.parallel_loop(half, chunk, step=_L, unroll=16)
    def _gather1(off):
        idx = spec_v[pl.ds(off, _L)]
        out_v[pl.ds(off, _L)] = plsc.load_gather(conv_v, [idx])

    cp_o1 = pltpu.async_copy(
        out_v.at[pl.ds(half, half)], out_hbm.at[pl.ds(base + half, half)],
        sem_s)
    cp_o0.wait()
    cp_o1.wait()


def kernel(species, coordinates, conv_tensor):
    shape = species.shape
    flat = species.reshape(-1)
    total = flat.shape[0]
    chunk = total // _NW
    conv_pad = jnp.pad(conv_tensor, (0, _LUT_PAD - conv_tensor.shape[0]))
    mesh = plsc.VectorSubcoreMesh(
        core_axis_name="c", subcore_axis_name="s", num_cores=_NC,
        num_subcores=_NS)
    out = pl.kernel(
        functools.partial(_lut_kernel, total),
        out_type=jax.ShapeDtypeStruct((total,), jnp.int32),
        mesh=mesh,
        scratch_types=[
            pltpu.VMEM((chunk,), jnp.int32),
            pltpu.VMEM((_LUT_PAD,), jnp.int32),
            pltpu.VMEM((chunk,), jnp.int32),
            pltpu.SemaphoreType.DMA,
            pltpu.SemaphoreType.DMA,
        ],
        compiler_params=pltpu.CompilerParams(needs_layout_passes=False),
    )(flat, conv_pad)
    return (out.reshape(shape), coordinates)
